# baseline (speedup 1.0000x reference)
import dataclasses
import functools

import jax
import jax.numpy as jnp
from jax import lax
from jax.experimental import pallas as pl
from jax.experimental.pallas import tpu as pltpu
from jax.experimental.pallas import tpu_sc as plsc

N = 10000
E = 320000
D = 128
NUM_DRUG = 216
NUM_RNA = N - NUM_DRUG

NC = 2
NS = 16
NW = NC * NS
CHUNK = 128
NCHUNKS = 2560
E_PAD = NCHUNKS * CHUNK
NCH_W = NCHUNKS // NW
NCH_S = NCHUNKS // NS
GROUP = 3
LANES = 16
BLK = 32
NBLK = NCH_S // BLK
MAXG = 12
CAPC = BLK + 4
CAP = CAPC * CHUNK

RLEN = 2560
ACC2 = 3072
RPS = ACC2 // NS
BIN = 2900
HIST = 10112
PAD_DST = 10008

_DOT_T = (((1,), (1,)), ((), ()))
_DOT_TL = (((0,), (0,)), ((), ()))
_PREC = jax.lax.Precision.HIGHEST

_SC_PARAMS = pltpu.CompilerParams()
_TC_PARAMS = pltpu.CompilerParams(vmem_limit_bytes=60 * 1024 * 1024)
if "needs_layout_passes" in pltpu.CompilerParams.__dataclass_fields__:
    _SC_PARAMS = dataclasses.replace(_SC_PARAMS, needs_layout_passes=False)


def _dot(a, b, dn):
    return lax.dot_general(a, b, dn, precision=_PREC,
                           preferred_element_type=jnp.float32)


def _sc_segment_sum(table, src2d, dst2d, lo0):
    mesh = plsc.VectorSubcoreMesh(core_axis_name="c", subcore_axis_name="s")

    @functools.partial(
        pl.kernel,
        out_type=jax.ShapeDtypeStruct((NC * ACC2, D), jnp.float32),
        mesh=mesh,
        scratch_types=[
            pltpu.VMEM((NCH_S, CHUNK), jnp.int32),
            pltpu.VMEM((NCH_S, CHUNK), jnp.int32),
            pltpu.VMEM((CAP,), jnp.int32),
            pltpu.VMEM((CAP,), jnp.int32),
            pltpu.VMEM((CAPC, CHUNK), jnp.int32),
            pltpu.VMEM((CHUNK, D), jnp.float32),
            pltpu.VMEM((CHUNK, D), jnp.float32),
            pltpu.VMEM((CHUNK, D), jnp.float32),
            pltpu.VMEM_SHARED((ACC2, D), jnp.float32),
            pltpu.SemaphoreType.DMA,
            pltpu.SemaphoreType.DMA,
            pltpu.SemaphoreType.DMA,
        ],
        compiler_params=_SC_PARAMS,
    )
    def k(table_hbm, src_hbm, dst_hbm, out_hbm,
          src_v, dst_v, csrc_f, cdst_f, cdst2, buf0, buf1, buf2, acc,
          sem0, sem1, sem2):
        bufs = (buf0, buf1, buf2)
        sems = (sem0, sem1, sem2)
        cid = lax.axis_index("c")
        sid = lax.axis_index("s")

        pltpu.sync_copy(src_hbm.at[pl.ds(sid * NCH_S, NCH_S)], src_v)
        pltpu.sync_copy(dst_hbm.at[pl.ds(sid * NCH_S, NCH_S)], dst_v)
        lo = lo0 + cid * RLEN

        zeros16 = jnp.zeros((LANES,), jnp.float32)

        @pl.loop(0, CHUNK)
        def _(i):
            for c in range(D // LANES):
                buf0[i, pl.ds(c * LANES, LANES)] = zeros16

        base = sid * RPS
        rem = RPS % CHUNK
        full = RPS - rem
        for r in range(RPS // CHUNK):
            pltpu.sync_copy(buf0, acc.at[pl.ds(base + r * CHUNK, CHUNK)])
        pltpu.sync_copy(buf0.at[pl.ds(0, rem)], acc.at[pl.ds(base + full, rem)])
        plsc.subcore_barrier()

        zeros16i = jnp.zeros((LANES,), jnp.int32)

        @pl.loop(0, NBLK)
        def _(blk):
            def scan_chunk(jj, w):
                for c in range(CHUNK // LANES):
                    d = dst_v[jj, pl.ds(c * LANES, LANES)]
                    s = src_v[jj, pl.ds(c * LANES, LANES)]
                    rel = d - lo
                    ok = (rel >= 0) & (rel < RLEN)
                    plsc.store_compressed(csrc_f.at[pl.ds(w, LANES)], s,
                                          mask=ok)
                    plsc.store_compressed(cdst_f.at[pl.ds(w, LANES)], rel,
                                          mask=ok)
                    w = w + jnp.sum(ok.astype(jnp.int32))
                return w

            w = lax.fori_loop(blk * BLK, (blk + 1) * BLK, scan_chunk, 0)

            for t in range(GROUP * CHUNK // LANES):
                csrc_f[pl.ds(w + t * LANES, LANES)] = zeros16i
                cdst_f[pl.ds(w + t * LANES, LANES)] = (
                    RLEN + (t % 4) * LANES + lax.iota(jnp.int32, LANES))
            ngroups = (w + GROUP * CHUNK) // (GROUP * CHUNK)
            nchunks = ngroups * GROUP

            @pl.loop(0, MAXG * GROUP)
            def _(q):
                @pl.when(q < nchunks)
                def _():
                    for c in range(CHUNK // LANES):
                        cdst2[q, pl.ds(c * LANES, LANES)] = (
                            cdst_f[pl.ds(q * CHUNK + c * LANES, LANES)])

            @pl.loop(0, MAXG)
            def _(g):
                @pl.when(g < ngroups)
                def _():
                    cps = [
                        pltpu.async_copy(
                            table_hbm.at[
                                csrc_f.at[pl.ds((g * GROUP + b) * CHUNK,
                                                CHUNK)]],
                            bufs[b], sems[b])
                        for b in range(GROUP)
                    ]
                    for b in range(GROUP):
                        cps[b].wait()
                        pltpu.sync_copy(bufs[b], acc.at[cdst2.at[g * GROUP + b]],
                                        add=True)

        plsc.subcore_barrier()
        out_base = cid * ACC2 + base
        for r in range(RPS // CHUNK):
            pltpu.sync_copy(acc.at[pl.ds(base + r * CHUNK, CHUNK)],
                            out_hbm.at[pl.ds(out_base + r * CHUNK, CHUNK)])
        pltpu.sync_copy(acc.at[pl.ds(base + full, rem)],
                        out_hbm.at[pl.ds(out_base + full, rem)])

    return k(table, src2d, dst2d)


def _sc_counts(dst2d):
    mesh = plsc.VectorSubcoreMesh(core_axis_name="c", subcore_axis_name="s")

    @functools.partial(
        pl.kernel,
        out_type=jax.ShapeDtypeStruct((NW, HIST), jnp.float32),
        mesh=mesh,
        scratch_types=[
            pltpu.VMEM((NCH_W, CHUNK), jnp.int32),
            pltpu.VMEM((HIST,), jnp.float32),
        ],
        compiler_params=_SC_PARAMS,
    )
    def k(dst_hbm, out_hbm, dst_v, hist):
        cid = lax.axis_index("c")
        sid = lax.axis_index("s")
        wid = sid * NC + cid
        pltpu.sync_copy(dst_hbm.at[pl.ds(wid * NCH_W, NCH_W)], dst_v)

        zeros16 = jnp.zeros((LANES,), jnp.float32)
        ones16 = jnp.ones((LANES,), jnp.float32)

        @pl.loop(0, HIST // LANES)
        def _(i):
            hist[pl.ds(i * LANES, LANES)] = zeros16

        @pl.loop(0, NCH_W)
        def _(j):
            for c in range(CHUNK // LANES):
                idx = dst_v[j, pl.ds(c * LANES, LANES)]
                plsc.addupdate_scatter(hist, [idx], ones16)

        pltpu.sync_copy(hist, out_hbm.at[wid])

    return k(dst2d)


def _tc_layer0(x, Wl1, Wr1):
    def body(x_ref, wl_ref, wr_ref, y_ref, z_ref):
        xx = x_ref[...]
        y_ref[...] = _dot(xx, wl_ref[...], _DOT_T)
        z_ref[...] = _dot(xx, wr_ref[...], _DOT_T)

    return pl.pallas_call(
        body,
        out_shape=(jax.ShapeDtypeStruct((N, D), jnp.float32),
                   jax.ShapeDtypeStruct((N, D), jnp.float32)),
        compiler_params=_TC_PARAMS,
    )(x, Wl1, Wr1)


def _mean_from(acca_ref, accb_ref, hist_ref):
    agg = jnp.concatenate(
        [acca_ref[0:RLEN], acca_ref[ACC2:ACC2 + RLEN],
         accb_ref[0:RLEN], accb_ref[ACC2:ACC2 + (N - 3 * RLEN)]], axis=0)
    ones = jnp.ones((NW, D), jnp.float32)
    cnt = _dot(hist_ref[...], ones, _DOT_TL)
    return agg / jnp.maximum(cnt[:N], 1.0)


def _tc_mid(acca, accb, hist, z1, b1, Wl2, Wr2):
    def body(acca_ref, accb_ref, hist_ref, z_ref, b_ref, wl_ref, wr_ref,
             y_ref, z2_ref):
        h = jnp.maximum(
            _mean_from(acca_ref, accb_ref, hist_ref) + z_ref[...] + b_ref[...],
            0.0)
        y_ref[...] = _dot(h, wl_ref[...], _DOT_T)
        z2_ref[...] = _dot(h, wr_ref[...], _DOT_T)

    return pl.pallas_call(
        body,
        out_shape=(jax.ShapeDtypeStruct((N, D), jnp.float32),
                   jax.ShapeDtypeStruct((N, D), jnp.float32)),
        compiler_params=_TC_PARAMS,
    )(acca, accb, hist, z1, b1, Wl2, Wr2)


def _tc_final(acca, accb, hist, z2, b2):
    def body(acca_ref, accb_ref, hist_ref, z_ref, b_ref, out_ref):
        h = _mean_from(acca_ref, accb_ref, hist_ref) + z_ref[...] + b_ref[...]
        out_ref[...] = _dot(h[:NUM_RNA], h[NUM_RNA:], _DOT_T)

    return pl.pallas_call(
        body,
        out_shape=jax.ShapeDtypeStruct((NUM_RNA, NUM_DRUG), jnp.float32),
        compiler_params=_TC_PARAMS,
    )(acca, accb, hist, z2, b2)


def kernel(x, edge_index, Wl1, Wr1, b1, Wl2, Wr2, b2):
    src = edge_index[0].astype(jnp.int32)
    dst = edge_index[1].astype(jnp.int32)
    src2d = jnp.concatenate(
        [src, jnp.zeros((E_PAD - E,), jnp.int32)]).reshape(NCHUNKS, CHUNK)
    dst2d = jnp.concatenate(
        [dst, jnp.full((E_PAD - E,), PAD_DST, jnp.int32)]).reshape(NCHUNKS, CHUNK)

    hist = _sc_counts(dst2d)
    y1, z1 = _tc_layer0(x, Wl1, Wr1)
    acc1a = _sc_segment_sum(y1, src2d, dst2d, 0)
    acc1b = _sc_segment_sum(y1, src2d, dst2d, 2 * RLEN)
    y2, z2 = _tc_mid(acc1a, acc1b, hist, z1, b1.reshape(1, D), Wl2, Wr2)
    acc2a = _sc_segment_sum(y2, src2d, dst2d, 0)
    acc2b = _sc_segment_sum(y2, src2d, dst2d, 2 * RLEN)
    return _tc_final(acc2a, acc2b, hist, z2, b2.reshape(1, D))


# speedup vs baseline: 1.4288x; 1.4288x over previous
import dataclasses
import functools

import jax
import jax.numpy as jnp
from jax import lax
from jax.experimental import pallas as pl
from jax.experimental.pallas import tpu as pltpu
from jax.experimental.pallas import tpu_sc as plsc

N = 10000
E = 320000
D = 128
NUM_DRUG = 216
NUM_RNA = N - NUM_DRUG

NC = 2
NS = 16
NW = NC * NS
CHUNK = 128
NCHUNKS = 2560
E_PAD = NCHUNKS * CHUNK
NCH_W = NCHUNKS // NW
NCH_S = NCHUNKS // NS
GROUP = 4
LANES = 16

RLEN = 2560
ACC2 = 3072
RPS = ACC2 // NS
BIN = 2900
HIST = 10112
PAD_DST = 10008

_DOT_T = (((1,), (1,)), ((), ()))
_DOT_TL = (((0,), (0,)), ((), ()))
_PREC = jax.lax.Precision.HIGHEST

_SC_PARAMS = pltpu.CompilerParams()
_TC_PARAMS = pltpu.CompilerParams(vmem_limit_bytes=60 * 1024 * 1024)
if "needs_layout_passes" in pltpu.CompilerParams.__dataclass_fields__:
    _SC_PARAMS = dataclasses.replace(_SC_PARAMS, needs_layout_passes=False)


def _dot(a, b, dn):
    return lax.dot_general(a, b, dn, precision=_PREC,
                           preferred_element_type=jnp.float32)


def _sc_segment_sum(table, src2d, dst2d, lo0):
    mesh = plsc.VectorSubcoreMesh(core_axis_name="c", subcore_axis_name="s")

    @functools.partial(
        pl.kernel,
        out_type=jax.ShapeDtypeStruct((NC * ACC2, D), jnp.float32),
        mesh=mesh,
        scratch_types=[
            pltpu.VMEM((NCH_S, CHUNK), jnp.int32),
            pltpu.VMEM((NCH_S, CHUNK), jnp.int32),
            pltpu.VMEM((CHUNK, D), jnp.float32),
            pltpu.VMEM((CHUNK, D), jnp.float32),
            pltpu.VMEM((CHUNK, D), jnp.float32),
            pltpu.VMEM((CHUNK, D), jnp.float32),
            pltpu.VMEM_SHARED((ACC2, D), jnp.float32),
            pltpu.SemaphoreType.DMA,
            pltpu.SemaphoreType.DMA,
            pltpu.SemaphoreType.DMA,
            pltpu.SemaphoreType.DMA,
            pltpu.SemaphoreType.DMA,
        ],
    )
    def k(table_hbm, src_hbm, dst_hbm, out_hbm,
          src_v, dst_v, buf0, buf1, buf2, buf3, acc,
          sem0, sem1, sem2, sem3, ssem):
        bufs = (buf0, buf1, buf2, buf3)
        sems = (sem0, sem1, sem2, sem3)
        cid = lax.axis_index("c")
        sid = lax.axis_index("s")

        pltpu.sync_copy(src_hbm.at[pl.ds(sid * NCH_S, NCH_S)], src_v)
        pltpu.sync_copy(dst_hbm.at[pl.ds(sid * NCH_S, NCH_S)], dst_v)
        lo = lo0 + cid * RLEN

        @pl.loop(0, NCH_S)
        def _(j):
            for c in range(CHUNK // LANES):
                d = dst_v[j, pl.ds(c * LANES, LANES)]
                rel = d - lo
                ok = (rel >= 0) & (rel < RLEN)
                dst_v[j, pl.ds(c * LANES, LANES)] = jnp.where(
                    ok, rel, RLEN + (d & (ACC2 - RLEN - 1)))

        zeros16 = jnp.zeros((LANES,), jnp.float32)

        @pl.loop(0, CHUNK)
        def _(i):
            for c in range(D // LANES):
                buf0[i, pl.ds(c * LANES, LANES)] = zeros16

        base = sid * RPS
        rem = RPS % CHUNK
        full = RPS - rem
        for r in range(RPS // CHUNK):
            pltpu.sync_copy(buf0, acc.at[pl.ds(base + r * CHUNK, CHUNK)])
        pltpu.sync_copy(buf0.at[pl.ds(0, rem)], acc.at[pl.ds(base + full, rem)])
        plsc.subcore_barrier()

        def start_gather(j, b):
            pltpu.async_copy(table_hbm.at[src_v.at[j]], bufs[b], sems[b])

        def wait_gather(j, b):
            pltpu.make_async_copy(table_hbm.at[src_v.at[j]], bufs[b],
                                  sems[b]).wait()

        def start_scatter(j, b):
            pltpu.async_copy(bufs[b], acc.at[dst_v.at[j]], ssem, add=True)

        def wait_scatter(j, b):
            pltpu.make_async_copy(bufs[b], acc.at[dst_v.at[j]], ssem).wait()

        QTR = NCH_S // 4
        for q in range(4):
            q0 = q * QTR
            for b in range(GROUP):
                start_gather(q0 + b, b)

            @pl.loop(q0, q0 + QTR - GROUP, step=GROUP)
            def _(j):
                for b in range(GROUP):
                    wait_gather(j + b, b)
                    start_scatter(j + b, b)
                for b in range(GROUP):
                    wait_scatter(j + b, b)
                    start_gather(j + GROUP + b, b)

            for b in range(GROUP):
                wait_gather(q0 + QTR - GROUP + b, b)
                start_scatter(q0 + QTR - GROUP + b, b)
            for b in range(GROUP):
                wait_scatter(q0 + QTR - GROUP + b, b)
            plsc.subcore_barrier()

        plsc.subcore_barrier()
        out_base = cid * ACC2 + base
        for r in range(RPS // CHUNK):
            pltpu.sync_copy(acc.at[pl.ds(base + r * CHUNK, CHUNK)],
                            out_hbm.at[pl.ds(out_base + r * CHUNK, CHUNK)])
        pltpu.sync_copy(acc.at[pl.ds(base + full, rem)],
                        out_hbm.at[pl.ds(out_base + full, rem)])

    return k(table, src2d, dst2d)


def _sc_counts(dst2d):
    mesh = plsc.VectorSubcoreMesh(core_axis_name="c", subcore_axis_name="s")

    @functools.partial(
        pl.kernel,
        out_type=jax.ShapeDtypeStruct((NW, HIST), jnp.float32),
        mesh=mesh,
        scratch_types=[
            pltpu.VMEM((NCH_W, CHUNK), jnp.int32),
            pltpu.VMEM((HIST,), jnp.float32),
        ],
        compiler_params=_SC_PARAMS,
    )
    def k(dst_hbm, out_hbm, dst_v, hist):
        cid = lax.axis_index("c")
        sid = lax.axis_index("s")
        wid = sid * NC + cid
        pltpu.sync_copy(dst_hbm.at[pl.ds(wid * NCH_W, NCH_W)], dst_v)

        zeros16 = jnp.zeros((LANES,), jnp.float32)
        ones16 = jnp.ones((LANES,), jnp.float32)

        @pl.loop(0, HIST // LANES)
        def _(i):
            hist[pl.ds(i * LANES, LANES)] = zeros16

        @pl.loop(0, NCH_W)
        def _(j):
            for c in range(CHUNK // LANES):
                idx = dst_v[j, pl.ds(c * LANES, LANES)]
                plsc.addupdate_scatter(hist, [idx], ones16)

        pltpu.sync_copy(hist, out_hbm.at[wid])

    return k(dst2d)


def _tc_layer0(x, Wl1, Wr1):
    def body(x_ref, wl_ref, wr_ref, y_ref, z_ref):
        xx = x_ref[...]
        y_ref[...] = _dot(xx, wl_ref[...], _DOT_T)
        z_ref[...] = _dot(xx, wr_ref[...], _DOT_T)

    return pl.pallas_call(
        body,
        out_shape=(jax.ShapeDtypeStruct((N, D), jnp.float32),
                   jax.ShapeDtypeStruct((N, D), jnp.float32)),
        compiler_params=_TC_PARAMS,
    )(x, Wl1, Wr1)


def _mean_from(acca_ref, accb_ref, hist_ref):
    agg = jnp.concatenate(
        [acca_ref[0:RLEN], acca_ref[ACC2:ACC2 + RLEN],
         accb_ref[0:RLEN], accb_ref[ACC2:ACC2 + (N - 3 * RLEN)]], axis=0)
    ones = jnp.ones((NW, D), jnp.float32)
    cnt = _dot(hist_ref[...], ones, _DOT_TL)
    return agg / jnp.maximum(cnt[:N], 1.0)


def _tc_mid(acca, accb, hist, z1, b1, Wl2, Wr2):
    def body(acca_ref, accb_ref, hist_ref, z_ref, b_ref, wl_ref, wr_ref,
             y_ref, z2_ref):
        h = jnp.maximum(
            _mean_from(acca_ref, accb_ref, hist_ref) + z_ref[...] + b_ref[...],
            0.0)
        y_ref[...] = _dot(h, wl_ref[...], _DOT_T)
        z2_ref[...] = _dot(h, wr_ref[...], _DOT_T)

    return pl.pallas_call(
        body,
        out_shape=(jax.ShapeDtypeStruct((N, D), jnp.float32),
                   jax.ShapeDtypeStruct((N, D), jnp.float32)),
        compiler_params=_TC_PARAMS,
    )(acca, accb, hist, z1, b1, Wl2, Wr2)


def _tc_final(acca, accb, hist, z2, b2):
    def body(acca_ref, accb_ref, hist_ref, z_ref, b_ref, out_ref):
        h = _mean_from(acca_ref, accb_ref, hist_ref) + z_ref[...] + b_ref[...]
        out_ref[...] = _dot(h[:NUM_RNA], h[NUM_RNA:], _DOT_T)

    return pl.pallas_call(
        body,
        out_shape=jax.ShapeDtypeStruct((NUM_RNA, NUM_DRUG), jnp.float32),
        compiler_params=_TC_PARAMS,
    )(acca, accb, hist, z2, b2)


def kernel(x, edge_index, Wl1, Wr1, b1, Wl2, Wr2, b2):
    src = edge_index[0].astype(jnp.int32)
    dst = edge_index[1].astype(jnp.int32)
    src2d = jnp.concatenate(
        [src, jnp.zeros((E_PAD - E,), jnp.int32)]).reshape(NCHUNKS, CHUNK)
    dst2d = jnp.concatenate(
        [dst, jnp.full((E_PAD - E,), PAD_DST, jnp.int32)]).reshape(NCHUNKS, CHUNK)

    hist = _sc_counts(dst2d)
    y1, z1 = _tc_layer0(x, Wl1, Wr1)
    acc1a = _sc_segment_sum(y1, src2d, dst2d, 0)
    acc1b = _sc_segment_sum(y1, src2d, dst2d, 2 * RLEN)
    y2, z2 = _tc_mid(acc1a, acc1b, hist, z1, b1.reshape(1, D), Wl2, Wr2)
    acc2a = _sc_segment_sum(y2, src2d, dst2d, 0)
    acc2b = _sc_segment_sum(y2, src2d, dst2d, 2 * RLEN)
    return _tc_final(acc2a, acc2b, hist, z2, b2.reshape(1, D))


# speedup vs baseline: 1.4398x; 1.0077x over previous
import dataclasses
import functools

import jax
import jax.numpy as jnp
from jax import lax
from jax.experimental import pallas as pl
from jax.experimental.pallas import tpu as pltpu
from jax.experimental.pallas import tpu_sc as plsc

N = 10000
E = 320000
D = 128
NUM_DRUG = 216
NUM_RNA = N - NUM_DRUG

NC = 2
NS = 16
NW = NC * NS
CHUNK = 128
NCHUNKS = 2560
E_PAD = NCHUNKS * CHUNK
NCH_W = NCHUNKS // NW
NCH_S = NCHUNKS // NS
GROUP = 4
LANES = 16

RLEN = 2560
ACC2 = 3072
RPS = ACC2 // NS
BIN = 2900
HIST = 10112
PAD_DST = 10008

_DOT_T = (((1,), (1,)), ((), ()))
_DOT_TL = (((0,), (0,)), ((), ()))
_PREC = jax.lax.Precision.HIGHEST

_SC_PARAMS = pltpu.CompilerParams()
_TC_PARAMS = pltpu.CompilerParams(vmem_limit_bytes=60 * 1024 * 1024)
if "needs_layout_passes" in pltpu.CompilerParams.__dataclass_fields__:
    _SC_PARAMS = dataclasses.replace(_SC_PARAMS, needs_layout_passes=False)


def _dot(a, b, dn):
    return lax.dot_general(a, b, dn, precision=_PREC,
                           preferred_element_type=jnp.float32)


def _sc_segment_sum(table, src2d, dst2d, lo0):
    mesh = plsc.VectorSubcoreMesh(core_axis_name="c", subcore_axis_name="s")

    @functools.partial(
        pl.kernel,
        out_type=jax.ShapeDtypeStruct((NC * ACC2, D), jnp.float32),
        mesh=mesh,
        scratch_types=[
            pltpu.VMEM((NCH_S, CHUNK), jnp.int32),
            pltpu.VMEM((NCH_S, CHUNK), jnp.int32),
            pltpu.VMEM((CHUNK, D), jnp.float32),
            pltpu.VMEM((CHUNK, D), jnp.float32),
            pltpu.VMEM((CHUNK, D), jnp.float32),
            pltpu.VMEM((CHUNK, D), jnp.float32),
            pltpu.VMEM_SHARED((ACC2, D), jnp.float32),
            pltpu.SemaphoreType.DMA,
            pltpu.SemaphoreType.DMA,
            pltpu.SemaphoreType.DMA,
            pltpu.SemaphoreType.DMA,
            pltpu.SemaphoreType.DMA,
        ],
    )
    def k(table_hbm, src_hbm, dst_hbm, out_hbm,
          src_v, dst_v, buf0, buf1, buf2, buf3, acc,
          sem0, sem1, sem2, sem3, ssem):
        bufs = (buf0, buf1, buf2, buf3)
        sems = (sem0, sem1, sem2, sem3)
        cid = lax.axis_index("c")
        sid = lax.axis_index("s")

        pltpu.sync_copy(src_hbm.at[pl.ds(sid * NCH_S, NCH_S)], src_v)
        pltpu.sync_copy(dst_hbm.at[pl.ds(sid * NCH_S, NCH_S)], dst_v)
        lo = lo0 + cid * RLEN

        @pl.loop(0, NCH_S)
        def _(j):
            for c in range(CHUNK // LANES):
                d = dst_v[j, pl.ds(c * LANES, LANES)]
                rel = d - lo
                ok = (rel >= 0) & (rel < RLEN)
                dst_v[j, pl.ds(c * LANES, LANES)] = jnp.where(
                    ok, rel, RLEN + (d & (ACC2 - RLEN - 1)))

        zeros16 = jnp.zeros((LANES,), jnp.float32)

        @pl.loop(0, CHUNK)
        def _(i):
            for c in range(D // LANES):
                buf0[i, pl.ds(c * LANES, LANES)] = zeros16

        base = sid * RPS
        rem = RPS % CHUNK
        full = RPS - rem
        for r in range(RPS // CHUNK):
            pltpu.sync_copy(buf0, acc.at[pl.ds(base + r * CHUNK, CHUNK)])
        pltpu.sync_copy(buf0.at[pl.ds(0, rem)], acc.at[pl.ds(base + full, rem)])
        plsc.subcore_barrier()

        def start_gather(j, b):
            pltpu.async_copy(table_hbm.at[src_v.at[j]], bufs[b], sems[b])

        def wait_gather(j, b):
            pltpu.make_async_copy(table_hbm.at[src_v.at[j]], bufs[b],
                                  sems[b]).wait()

        def start_scatter(j, b):
            pltpu.async_copy(bufs[b], acc.at[dst_v.at[j]], ssem, add=True)

        def wait_scatter(j, b):
            pltpu.make_async_copy(bufs[b], acc.at[dst_v.at[j]], ssem).wait()

        for b in range(GROUP):
            start_gather(b, b)

        @pl.loop(0, NCH_S - GROUP, step=GROUP)
        def _(j):
            for b in range(GROUP):
                wait_gather(j + b, b)
                start_scatter(j + b, b)
            for b in range(GROUP):
                wait_scatter(j + b, b)
                start_gather(j + GROUP + b, b)

        for b in range(GROUP):
            wait_gather(NCH_S - GROUP + b, b)
            start_scatter(NCH_S - GROUP + b, b)
        for b in range(GROUP):
            wait_scatter(NCH_S - GROUP + b, b)

        plsc.subcore_barrier()
        out_base = cid * ACC2 + base
        for r in range(RPS // CHUNK):
            pltpu.sync_copy(acc.at[pl.ds(base + r * CHUNK, CHUNK)],
                            out_hbm.at[pl.ds(out_base + r * CHUNK, CHUNK)])
        pltpu.sync_copy(acc.at[pl.ds(base + full, rem)],
                        out_hbm.at[pl.ds(out_base + full, rem)])

    return k(table, src2d, dst2d)


def _sc_counts(dst2d):
    mesh = plsc.VectorSubcoreMesh(core_axis_name="c", subcore_axis_name="s")

    @functools.partial(
        pl.kernel,
        out_type=jax.ShapeDtypeStruct((NW, HIST), jnp.float32),
        mesh=mesh,
        scratch_types=[
            pltpu.VMEM((NCH_W, CHUNK), jnp.int32),
            pltpu.VMEM((HIST,), jnp.float32),
        ],
        compiler_params=_SC_PARAMS,
    )
    def k(dst_hbm, out_hbm, dst_v, hist):
        cid = lax.axis_index("c")
        sid = lax.axis_index("s")
        wid = sid * NC + cid
        pltpu.sync_copy(dst_hbm.at[pl.ds(wid * NCH_W, NCH_W)], dst_v)

        zeros16 = jnp.zeros((LANES,), jnp.float32)
        ones16 = jnp.ones((LANES,), jnp.float32)

        @pl.loop(0, HIST // LANES)
        def _(i):
            hist[pl.ds(i * LANES, LANES)] = zeros16

        @pl.loop(0, NCH_W)
        def _(j):
            for c in range(CHUNK // LANES):
                idx = dst_v[j, pl.ds(c * LANES, LANES)]
                plsc.addupdate_scatter(hist, [idx], ones16)

        pltpu.sync_copy(hist, out_hbm.at[wid])

    return k(dst2d)


def _tc_layer0(x, Wl1, Wr1):
    def body(x_ref, wl_ref, wr_ref, y_ref, z_ref):
        xx = x_ref[...]
        y_ref[...] = _dot(xx, wl_ref[...], _DOT_T)
        z_ref[...] = _dot(xx, wr_ref[...], _DOT_T)

    return pl.pallas_call(
        body,
        out_shape=(jax.ShapeDtypeStruct((N, D), jnp.float32),
                   jax.ShapeDtypeStruct((N, D), jnp.float32)),
        compiler_params=_TC_PARAMS,
    )(x, Wl1, Wr1)


def _mean_from(acca_ref, accb_ref, hist_ref):
    agg = jnp.concatenate(
        [acca_ref[0:RLEN], acca_ref[ACC2:ACC2 + RLEN],
         accb_ref[0:RLEN], accb_ref[ACC2:ACC2 + (N - 3 * RLEN)]], axis=0)
    ones = jnp.ones((NW, D), jnp.float32)
    cnt = _dot(hist_ref[...], ones, _DOT_TL)
    return agg / jnp.maximum(cnt[:N], 1.0)


def _tc_mid(acca, accb, hist, z1, b1, Wl2, Wr2):
    def body(acca_ref, accb_ref, hist_ref, z_ref, b_ref, wl_ref, wr_ref,
             y_ref, z2_ref):
        h = jnp.maximum(
            _mean_from(acca_ref, accb_ref, hist_ref) + z_ref[...] + b_ref[...],
            0.0)
        y_ref[...] = _dot(h, wl_ref[...], _DOT_T)
        z2_ref[...] = _dot(h, wr_ref[...], _DOT_T)

    return pl.pallas_call(
        body,
        out_shape=(jax.ShapeDtypeStruct((N, D), jnp.float32),
                   jax.ShapeDtypeStruct((N, D), jnp.float32)),
        compiler_params=_TC_PARAMS,
    )(acca, accb, hist, z1, b1, Wl2, Wr2)


def _tc_final(acca, accb, hist, z2, b2):
    def body(acca_ref, accb_ref, hist_ref, z_ref, b_ref, out_ref):
        h = _mean_from(acca_ref, accb_ref, hist_ref) + z_ref[...] + b_ref[...]
        out_ref[...] = _dot(h[:NUM_RNA], h[NUM_RNA:], _DOT_T)

    return pl.pallas_call(
        body,
        out_shape=jax.ShapeDtypeStruct((NUM_RNA, NUM_DRUG), jnp.float32),
        compiler_params=_TC_PARAMS,
    )(acca, accb, hist, z2, b2)


def kernel(x, edge_index, Wl1, Wr1, b1, Wl2, Wr2, b2):
    src = edge_index[0].astype(jnp.int32)
    dst = edge_index[1].astype(jnp.int32)
    src2d = jnp.concatenate(
        [src, jnp.zeros((E_PAD - E,), jnp.int32)]).reshape(NCHUNKS, CHUNK)
    dst2d = jnp.concatenate(
        [dst, jnp.full((E_PAD - E,), PAD_DST, jnp.int32)]).reshape(NCHUNKS, CHUNK)

    hist = _sc_counts(dst2d)
    y1, z1 = _tc_layer0(x, Wl1, Wr1)
    acc1a = _sc_segment_sum(y1, src2d, dst2d, 0)
    acc1b = _sc_segment_sum(y1, src2d, dst2d, 2 * RLEN)
    y2, z2 = _tc_mid(acc1a, acc1b, hist, z1, b1.reshape(1, D), Wl2, Wr2)
    acc2a = _sc_segment_sum(y2, src2d, dst2d, 0)
    acc2b = _sc_segment_sum(y2, src2d, dst2d, 2 * RLEN)
    return _tc_final(acc2a, acc2b, hist, z2, b2.reshape(1, D))
